# baseline (speedup 1.0000x reference)
import functools

import jax
import jax.numpy as jnp
from jax import lax
from jax.experimental import pallas as pl
from jax.experimental.pallas import tpu as pltpu
from jax.experimental.pallas import tpu_sc as plsc

EMBED_DIM = 64
CHUNK = 128


@functools.lru_cache(maxsize=None)
def _gather_fn(batch: int, num_users: int, num_movies: int):
    info = plsc.get_sparse_core_info()
    nw = info.num_cores * info.num_subcores
    rows_per_w = batch // nw
    n_chunks = rows_per_w // CHUNK
    assert rows_per_w % CHUNK == 0 and batch % nw == 0

    mesh = plsc.VectorSubcoreMesh(core_axis_name="c", subcore_axis_name="s")

    @functools.partial(
        pl.kernel,
        mesh=mesh,
        compiler_params=pltpu.CompilerParams(use_tc_tiling_on_sc=False),
        out_type=[
            jax.ShapeDtypeStruct((batch, EMBED_DIM), jnp.float32),
            jax.ShapeDtypeStruct((batch, EMBED_DIM), jnp.float32),
        ],
        scratch_types=[
            pltpu.VMEM((n_chunks, CHUNK), jnp.int32),
            pltpu.VMEM((rows_per_w, EMBED_DIM), jnp.float32),
            pltpu.VMEM((n_chunks, CHUNK), jnp.int32),
            pltpu.VMEM((rows_per_w, EMBED_DIM), jnp.float32),
            pltpu.SemaphoreType.DMA,
            pltpu.SemaphoreType.DMA,
        ],
    )
    def gather(user_table, user_ids, movie_table, movie_ids, u_out, m_out,
               uidx_v, urows_v, midx_v, mrows_v, sem_u, sem_m):
        wid = lax.axis_index("s") * info.num_cores + lax.axis_index("c")
        base = wid * rows_per_w
        pltpu.sync_copy(user_ids.at[pl.ds(wid * n_chunks, n_chunks)], uidx_v)
        pltpu.sync_copy(movie_ids.at[pl.ds(wid * n_chunks, n_chunks)], midx_v)
        ucopies = [
            pltpu.async_copy(
                user_table.at[uidx_v.at[j]],
                urows_v.at[pl.ds(j * CHUNK, CHUNK)],
                sem_u,
            )
            for j in range(n_chunks)
        ]
        mcopies = [
            pltpu.async_copy(
                movie_table.at[midx_v.at[j]],
                mrows_v.at[pl.ds(j * CHUNK, CHUNK)],
                sem_m,
            )
            for j in range(n_chunks)
        ]
        for c in ucopies:
            c.wait()
        pltpu.sync_copy(urows_v, u_out.at[pl.ds(base, rows_per_w)])
        for c in mcopies:
            c.wait()
        pltpu.sync_copy(mrows_v, m_out.at[pl.ds(base, rows_per_w)])

    return gather


def _mlp_body(u_ref, m_ref, w1u_ref, w1m_ref, b1_ref, w2_ref, b2_ref,
              w3_ref, b3_ref, out_ref):
    h = u_ref[:] @ w1u_ref[:] + m_ref[:] @ w1m_ref[:] + b1_ref[:]
    h = jnp.maximum(h, 0.0)
    h = jnp.maximum(h @ w2_ref[:] + b2_ref[:], 0.0)
    out_ref[:] = jnp.sum(h * w3_ref[:], axis=1) + b3_ref[0, 0]


def _mlp(u, m, w1u, w1m, b1r, w2, b2r, w3r, b3r, block_b: int, interpret=False):
    batch = u.shape[0]
    h1 = w1u.shape[1]
    h2 = w2.shape[1]
    grid = (batch // block_b,)
    return pl.pallas_call(
        _mlp_body,
        grid=grid,
        in_specs=[
            pl.BlockSpec((block_b, EMBED_DIM), lambda i: (i, 0)),
            pl.BlockSpec((block_b, EMBED_DIM), lambda i: (i, 0)),
            pl.BlockSpec((EMBED_DIM, h1), lambda i: (0, 0)),
            pl.BlockSpec((EMBED_DIM, h1), lambda i: (0, 0)),
            pl.BlockSpec((1, h1), lambda i: (0, 0)),
            pl.BlockSpec((h1, h2), lambda i: (0, 0)),
            pl.BlockSpec((1, h2), lambda i: (0, 0)),
            pl.BlockSpec((1, h2), lambda i: (0, 0)),
            pl.BlockSpec((1, 1), lambda i: (0, 0)),
        ],
        out_specs=pl.BlockSpec((block_b,), lambda i: (i,)),
        out_shape=jax.ShapeDtypeStruct((batch,), jnp.float32),
        interpret=interpret,
    )(u, m, w1u, w1m, b1r, w2, b2r, w3r, b3r)


def kernel(user_ids, movie_ids, user_table, movie_table, W1, b1, W2, b2, W3, b3):
    batch = user_ids.shape[0]
    gather = _gather_fn(batch, user_table.shape[0], movie_table.shape[0])
    uids = user_ids.astype(jnp.int32).reshape(batch // CHUNK, CHUNK)
    mids = movie_ids.astype(jnp.int32).reshape(batch // CHUNK, CHUNK)
    u, m = gather(user_table, uids, movie_table, mids)
    w1u = W1[:EMBED_DIM]
    w1m = W1[EMBED_DIM:]
    return _mlp(
        u, m, w1u, w1m,
        b1.reshape(1, -1), W2, b2.reshape(1, -1),
        W3.reshape(1, -1), b3.reshape(1, 1),
        block_b=2048,
    )


# speedup vs baseline: 1.5735x; 1.5735x over previous
import functools

import jax
import jax.numpy as jnp
from jax import lax
from jax.experimental import pallas as pl
from jax.experimental.pallas import tpu as pltpu
from jax.experimental.pallas import tpu_sc as plsc

EMBED_DIM = 64


@functools.lru_cache(maxsize=None)
def _gather_fn(batch: int, num_rows: int):
    info = plsc.get_sparse_core_info()
    nw = info.num_cores * info.num_subcores
    rows_per_w = batch // nw
    assert batch % nw == 0

    mesh = plsc.VectorSubcoreMesh(core_axis_name="c", subcore_axis_name="s")

    @functools.partial(
        pl.kernel,
        mesh=mesh,
        out_type=jax.ShapeDtypeStruct((batch, EMBED_DIM), jnp.float32),
        scratch_types=[
            pltpu.VMEM((rows_per_w, EMBED_DIM), jnp.float32),
            pltpu.VMEM_SHARED((batch // 2,), jnp.int32),
            pltpu.SMEM((rows_per_w,), jnp.int32),
            pltpu.SemaphoreType.DMA,
        ],
    )
    def gather(table, ids, out, rows_v, idx_sh, idx_s, sem):
        wid = lax.axis_index("s") * info.num_cores + lax.axis_index("c")
        base = wid * rows_per_w
        sub = lax.axis_index("s") * rows_per_w
        pltpu.sync_copy(ids.at[pl.ds(base, rows_per_w)],
                        idx_sh.at[pl.ds(sub, rows_per_w)])
        pltpu.sync_copy(idx_sh.at[pl.ds(sub, rows_per_w)], idx_s)

        def fire(j, _):
            pltpu.make_async_copy(
                table.at[pl.ds(idx_s[j], 1)],
                rows_v.at[pl.ds(j, 1)],
                sem,
            ).start()
            return _

        lax.fori_loop(0, rows_per_w, fire, 0)

        def drain(j, _):
            pltpu.make_async_copy(
                table.at[pl.ds(0, 1)],
                rows_v.at[pl.ds(0, 1)],
                sem,
            ).wait()
            return _

        lax.fori_loop(0, rows_per_w, drain, 0)
        pltpu.sync_copy(rows_v, out.at[pl.ds(base, rows_per_w)])

    return gather


def _mlp_body(u_ref, m_ref, w1u_ref, w1m_ref, b1_ref, w2_ref, b2_ref,
              w3_ref, b3_ref, out_ref):
    h = u_ref[:] @ w1u_ref[:] + m_ref[:] @ w1m_ref[:] + b1_ref[:]
    h = jnp.maximum(h, 0.0)
    h = jnp.maximum(h @ w2_ref[:] + b2_ref[:], 0.0)
    out_ref[:] = jnp.sum(h * w3_ref[:], axis=1) + b3_ref[0, 0]


def _mlp(u, m, w1u, w1m, b1r, w2, b2r, w3r, b3r, block_b: int, interpret=False):
    batch = u.shape[0]
    h1 = w1u.shape[1]
    h2 = w2.shape[1]
    grid = (batch // block_b,)
    return pl.pallas_call(
        _mlp_body,
        grid=grid,
        in_specs=[
            pl.BlockSpec((block_b, EMBED_DIM), lambda i: (i, 0)),
            pl.BlockSpec((block_b, EMBED_DIM), lambda i: (i, 0)),
            pl.BlockSpec((EMBED_DIM, h1), lambda i: (0, 0)),
            pl.BlockSpec((EMBED_DIM, h1), lambda i: (0, 0)),
            pl.BlockSpec((1, h1), lambda i: (0, 0)),
            pl.BlockSpec((h1, h2), lambda i: (0, 0)),
            pl.BlockSpec((1, h2), lambda i: (0, 0)),
            pl.BlockSpec((1, h2), lambda i: (0, 0)),
            pl.BlockSpec((1, 1), lambda i: (0, 0)),
        ],
        out_specs=pl.BlockSpec((block_b,), lambda i: (i,)),
        out_shape=jax.ShapeDtypeStruct((batch,), jnp.float32),
        interpret=interpret,
    )(u, m, w1u, w1m, b1r, w2, b2r, w3r, b3r)


def kernel(user_ids, movie_ids, user_table, movie_table, W1, b1, W2, b2, W3, b3):
    batch = user_ids.shape[0]
    gather_u = _gather_fn(batch, user_table.shape[0])
    gather_m = _gather_fn(batch, movie_table.shape[0])
    u = gather_u(user_table, user_ids.astype(jnp.int32))
    m = gather_m(movie_table, movie_ids.astype(jnp.int32))
    return _mlp(
        u, m, W1[:EMBED_DIM], W1[EMBED_DIM:],
        b1.reshape(1, -1), W2, b2.reshape(1, -1),
        W3.reshape(1, -1), b3.reshape(1, 1),
        block_b=2048,
    )
